# baseline (speedup 1.0000x reference)
import functools

import jax
import jax.numpy as jnp
from jax import lax
from jax.experimental import pallas as pl
from jax.experimental.pallas import tpu as pltpu
from jax.experimental.pallas import tpu_sc as plsc

_NC = 2
_NS = 16
_NW = _NC * _NS
_W = 128


def _gather_kernel(idx_hbm, table_hbm, out_hbm, idx_v, rows_v, sem):
    g_per_w = idx_v.shape[0]
    wid = lax.axis_index("s") * _NC + lax.axis_index("c")
    row0 = wid * g_per_w
    pltpu.sync_copy(idx_hbm.at[pl.ds(row0, g_per_w)], idx_v)

    def body(g, carry):
        pltpu.async_copy(table_hbm.at[idx_v.at[g]], rows_v, sem).wait()
        pltpu.sync_copy(rows_v, out_hbm.at[pl.ds((row0 + g) * _W, _W)])
        return carry

    lax.fori_loop(0, g_per_w, body, 0)


def kernel(input, table):
    batch, hist = input.shape
    _, embed_dim = table.shape
    n = batch * hist
    assert n % (_NW * _W) == 0
    g_per_w = n // (_NW * _W)
    idx2d = input.reshape(n // _W, _W)

    run = functools.partial(
        pl.kernel,
        out_type=jax.ShapeDtypeStruct((n, embed_dim), table.dtype),
        mesh=plsc.VectorSubcoreMesh(core_axis_name="c", subcore_axis_name="s"),
        scratch_types=[
            pltpu.VMEM((g_per_w, _W), jnp.int32),
            pltpu.VMEM((_W, embed_dim), jnp.float32),
            pltpu.SemaphoreType.DMA,
        ],
        compiler_params=pltpu.CompilerParams(use_tc_tiling_on_sc=False),
    )(_gather_kernel)

    out = run(idx2d, table)
    return out.reshape(batch, hist, embed_dim)


# speedup vs baseline: 1.3747x; 1.3747x over previous
import functools

import jax
import jax.numpy as jnp
from jax import lax
from jax.experimental import pallas as pl
from jax.experimental.pallas import tpu as pltpu
from jax.experimental.pallas import tpu_sc as plsc

_NC = 2
_NS = 16
_NW = _NC * _NS
_W = 128
_NBUF = 4


def _gather_kernel(idx_hbm, table_hbm, out_hbm, idx_v, bufs, sem_g, sem_w):
    g_total = idx_v.shape[0]
    nr = g_total // _NBUF
    wid = lax.axis_index("s") * _NC + lax.axis_index("c")
    row0 = wid * g_total
    pltpu.sync_copy(idx_hbm.at[pl.ds(row0, g_total)], idx_v)

    def fire_g(slot, g):
        pltpu.async_copy(table_hbm.at[idx_v.at[g]], bufs.at[slot], sem_g.at[slot])

    def wait_g(slot, g):
        pltpu.make_async_copy(
            table_hbm.at[idx_v.at[g]], bufs.at[slot], sem_g.at[slot]
        ).wait()

    def out_slice(g):
        return out_hbm.at[pl.ds((row0 + g) * _W, _W)]

    def fire_w(slot, g):
        pltpu.async_copy(bufs.at[slot], out_slice(g), sem_w.at[slot])

    def wait_w(slot, g):
        pltpu.make_async_copy(bufs.at[slot], out_slice(g), sem_w.at[slot]).wait()

    for b in range(_NBUF):
        fire_g(b, b)
    for b in range(_NBUF):
        wait_g(b, b)
        fire_w(b, b)
    for b in range(_NBUF):
        fire_g(_NBUF + b, _NBUF + b)

    @pl.loop(1, nr - 1, step=2)
    def _steady(r):
        for b in range(_NBUF):
            wait_g(_NBUF + b, r * _NBUF + b)
            fire_w(_NBUF + b, r * _NBUF + b)
        for b in range(_NBUF):
            wait_w(b, (r - 1) * _NBUF + b)
            fire_g(b, (r + 1) * _NBUF + b)
        for b in range(_NBUF):
            wait_g(b, (r + 1) * _NBUF + b)
            fire_w(b, (r + 1) * _NBUF + b)
        for b in range(_NBUF):
            wait_w(_NBUF + b, r * _NBUF + b)
            fire_g(_NBUF + b, (r + 2) * _NBUF + b)

    for b in range(_NBUF):
        wait_g(_NBUF + b, (nr - 1) * _NBUF + b)
        fire_w(_NBUF + b, (nr - 1) * _NBUF + b)
    for b in range(_NBUF):
        wait_w(b, (nr - 2) * _NBUF + b)
    for b in range(_NBUF):
        wait_w(_NBUF + b, (nr - 1) * _NBUF + b)


def kernel(input, table):
    batch, hist = input.shape
    _, embed_dim = table.shape
    n = batch * hist
    assert n % (_NW * _W * 2 * _NBUF) == 0
    g_per_w = n // (_NW * _W)
    idx2d = input.reshape(n // _W, _W)

    run = functools.partial(
        pl.kernel,
        out_type=jax.ShapeDtypeStruct((n, embed_dim), table.dtype),
        mesh=plsc.VectorSubcoreMesh(core_axis_name="c", subcore_axis_name="s"),
        scratch_types=[
            pltpu.VMEM((g_per_w, _W), jnp.int32),
            pltpu.VMEM((2 * _NBUF, _W, embed_dim), jnp.float32),
            pltpu.SemaphoreType.DMA((2 * _NBUF,)),
            pltpu.SemaphoreType.DMA((2 * _NBUF,)),
        ],
        compiler_params=pltpu.CompilerParams(use_tc_tiling_on_sc=False),
    )(_gather_kernel)

    out = run(idx2d, table)
    return out.reshape(batch, hist, embed_dim)
